# Initial kernel scaffold; baseline (speedup 1.0000x reference)
import jax
import jax.numpy as jnp
from jax.experimental import pallas as pl


def kernel(x, e, u, W, b):
    raise NotImplementedError("write your pallas kernel here")


# speedup vs baseline: 88.6753x; 88.6753x over previous
import jax
import jax.numpy as jnp
from jax.experimental import pallas as pl

_T = 2048
_C = 512


def _sample_body(x_ref, ut_ref, w_ref, b_ref, post_ref, sel_ref):
    xb = x_ref[0]
    w8 = w_ref[...]
    h = jax.lax.dot_general(w8, xb, (((1,), (0,)), ((), ())),
                            preferred_element_type=jnp.float32)
    h = h + b_ref[...]
    z0 = h[0:1, :] / 10.0
    z1 = h[1:2, :] / 10.0
    m = jnp.maximum(z0, z1)
    e0 = jnp.exp(z0 - m)
    e1 = jnp.exp(z1 - m)
    s = e0 + e1
    p0 = e0 / s
    p1 = e1 / s
    post_ref[0, 0:1, :] = p0
    post_ref[0, 1:2, :] = p1
    eps = 1e-20
    l0 = jnp.log(p0)
    l1 = jnp.log(p1)
    u0 = ut_ref[0, 0:1, :]
    u1 = ut_ref[0, 1:2, :]
    g0 = -jnp.log(-jnp.log(u0 + eps) + eps)
    g1 = -jnp.log(-jnp.log(u1 + eps) + eps)
    zz0 = (l0 + g0) / 0.8
    zz1 = (l1 + g1) / 0.8
    mm = jnp.maximum(zz0, zz1)
    ee0 = jnp.exp(zz0 - mm)
    ee1 = jnp.exp(zz1 - mm)
    ss = ee0 + ee1
    y0 = ee0 / ss
    y1 = ee1 / ss
    sel_ref[0, 0:1, :] = jnp.where(y1 > y0, 1.0, 0.0).astype(jnp.float32)


def _median5(z0, z1, z2, z3, z4):
    a0 = jnp.minimum(z0, z1)
    a1 = jnp.maximum(z0, z1)
    a3 = jnp.minimum(z3, z4)
    a4 = jnp.maximum(z3, z4)
    b3 = jnp.maximum(a0, a3)
    b1 = jnp.minimum(a1, a4)
    c1 = jnp.minimum(b1, z2)
    c2 = jnp.maximum(b1, z2)
    d2 = jnp.minimum(c2, b3)
    return jnp.maximum(c1, d2)


def _medpool(v):
    T = _T
    s0 = jnp.concatenate([v[:, 2:3], v[:, 1:2], v[:, :T - 2]], axis=1)
    s1 = jnp.concatenate([v[:, 1:2], v[:, :T - 1]], axis=1)
    s3 = jnp.concatenate([v[:, 1:], v[:, T - 2:T - 1]], axis=1)
    s4 = jnp.concatenate([v[:, 2:], v[:, T - 2:T - 1], v[:, T - 3:T - 2]],
                         axis=1)
    return _median5(s0, s1, v, s3, s4)


def _mask_body(e_ref, sel_ref, mask_ref):
    v = e_ref[0] * sel_ref[0]
    v = _medpool(v)
    v = _medpool(v)
    v = _medpool(v)
    mask_ref[0] = v


def kernel(x, e, u, W, b):
    B, C, T = x.shape
    w8 = jnp.zeros((8, C), jnp.float32).at[:2].set(W)
    b8 = jnp.zeros((8, 1), jnp.float32).at[:2, 0].set(b)
    ut = jnp.transpose(u, (0, 2, 1))

    post_bt, sel = pl.pallas_call(
        _sample_body,
        grid=(B,),
        in_specs=[
            pl.BlockSpec((1, C, T), lambda i: (i, 0, 0)),
            pl.BlockSpec((1, 2, T), lambda i: (i, 0, 0)),
            pl.BlockSpec((8, C), lambda i: (0, 0)),
            pl.BlockSpec((8, 1), lambda i: (0, 0)),
        ],
        out_specs=[
            pl.BlockSpec((1, 2, T), lambda i: (i, 0, 0)),
            pl.BlockSpec((1, 1, T), lambda i: (i, 0, 0)),
        ],
        out_shape=[
            jax.ShapeDtypeStruct((B, 2, T), jnp.float32),
            jax.ShapeDtypeStruct((B, 1, T), jnp.float32),
        ],
    )(x, ut, w8, b8)

    Cb = 128
    mask = pl.pallas_call(
        _mask_body,
        grid=(B, C // Cb),
        in_specs=[
            pl.BlockSpec((1, Cb, T), lambda i, j: (i, j, 0)),
            pl.BlockSpec((1, 1, T), lambda i, j: (i, 0, 0)),
        ],
        out_specs=pl.BlockSpec((1, Cb, T), lambda i, j: (i, j, 0)),
        out_shape=jax.ShapeDtypeStruct((B, C, T), jnp.float32),
    )(e, sel)

    posterior = jnp.transpose(post_bt, (0, 2, 1))
    return posterior, mask


# speedup vs baseline: 88.9278x; 1.0028x over previous
import jax
import jax.numpy as jnp
from jax.experimental import pallas as pl

_T = 2048
_C = 512


def _sample_body(x_ref, ut_ref, w_ref, b_ref, post_ref, sel_ref):
    xb = x_ref[0]
    w8 = w_ref[...]
    h = jax.lax.dot_general(w8, xb, (((1,), (0,)), ((), ())),
                            preferred_element_type=jnp.float32)
    h = h + b_ref[...]
    z0 = h[0:1, :] / 10.0
    z1 = h[1:2, :] / 10.0
    m = jnp.maximum(z0, z1)
    e0 = jnp.exp(z0 - m)
    e1 = jnp.exp(z1 - m)
    s = e0 + e1
    p0 = e0 / s
    p1 = e1 / s
    post_ref[0, 0:1, :] = p0
    post_ref[0, 1:2, :] = p1
    eps = 1e-20
    l0 = jnp.log(p0)
    l1 = jnp.log(p1)
    u0 = ut_ref[0, 0:1, :]
    u1 = ut_ref[0, 1:2, :]
    g0 = -jnp.log(-jnp.log(u0 + eps) + eps)
    g1 = -jnp.log(-jnp.log(u1 + eps) + eps)
    zz0 = (l0 + g0) / 0.8
    zz1 = (l1 + g1) / 0.8
    mm = jnp.maximum(zz0, zz1)
    ee0 = jnp.exp(zz0 - mm)
    ee1 = jnp.exp(zz1 - mm)
    ss = ee0 + ee1
    y0 = ee0 / ss
    y1 = ee1 / ss
    sel_ref[0, 0:1, :] = jnp.where(y1 > y0, 1.0, 0.0).astype(jnp.float32)


def _median5(z0, z1, z2, z3, z4):
    a0 = jnp.minimum(z0, z1)
    a1 = jnp.maximum(z0, z1)
    a3 = jnp.minimum(z3, z4)
    a4 = jnp.maximum(z3, z4)
    b3 = jnp.maximum(a0, a3)
    b1 = jnp.minimum(a1, a4)
    c1 = jnp.minimum(b1, z2)
    c2 = jnp.maximum(b1, z2)
    d2 = jnp.minimum(c2, b3)
    return jnp.maximum(c1, d2)


def _medpool(v):
    T = _T
    s0 = jnp.concatenate([v[:, 2:3], v[:, 1:2], v[:, :T - 2]], axis=1)
    s1 = jnp.concatenate([v[:, 1:2], v[:, :T - 1]], axis=1)
    s3 = jnp.concatenate([v[:, 1:], v[:, T - 2:T - 1]], axis=1)
    s4 = jnp.concatenate([v[:, 2:], v[:, T - 2:T - 1], v[:, T - 3:T - 2]],
                         axis=1)
    return _median5(s0, s1, v, s3, s4)


def _mask_body(e_ref, sel_ref, mask_ref):
    v = e_ref[0] * sel_ref[0]
    v = _medpool(v)
    v = _medpool(v)
    v = _medpool(v)
    mask_ref[0] = v


def kernel(x, e, u, W, b):
    B, C, T = x.shape
    w8 = jnp.zeros((8, C), jnp.float32).at[:2].set(W)
    b8 = jnp.zeros((8, 1), jnp.float32).at[:2, 0].set(b)
    ut = jnp.transpose(u, (0, 2, 1))

    post_bt, sel = pl.pallas_call(
        _sample_body,
        grid=(B,),
        in_specs=[
            pl.BlockSpec((1, C, T), lambda i: (i, 0, 0)),
            pl.BlockSpec((1, 2, T), lambda i: (i, 0, 0)),
            pl.BlockSpec((8, C), lambda i: (0, 0)),
            pl.BlockSpec((8, 1), lambda i: (0, 0)),
        ],
        out_specs=[
            pl.BlockSpec((1, 2, T), lambda i: (i, 0, 0)),
            pl.BlockSpec((1, 1, T), lambda i: (i, 0, 0)),
        ],
        out_shape=[
            jax.ShapeDtypeStruct((B, 2, T), jnp.float32),
            jax.ShapeDtypeStruct((B, 1, T), jnp.float32),
        ],
    )(x, ut, w8, b8)

    Cb = 256
    mask = pl.pallas_call(
        _mask_body,
        grid=(B, C // Cb),
        in_specs=[
            pl.BlockSpec((1, Cb, T), lambda i, j: (i, j, 0)),
            pl.BlockSpec((1, 1, T), lambda i, j: (i, 0, 0)),
        ],
        out_specs=pl.BlockSpec((1, Cb, T), lambda i, j: (i, j, 0)),
        out_shape=jax.ShapeDtypeStruct((B, C, T), jnp.float32),
    )(e, sel)

    posterior = jnp.transpose(post_bt, (0, 2, 1))
    return posterior, mask


# speedup vs baseline: 158.2633x; 1.7797x over previous
import jax
import jax.numpy as jnp
from jax.experimental import pallas as pl

_T = 2048


def _median5(z0, z1, z2, z3, z4):
    a0 = jnp.minimum(z0, z1)
    a1 = jnp.maximum(z0, z1)
    a3 = jnp.minimum(z3, z4)
    a4 = jnp.maximum(z3, z4)
    b3 = jnp.maximum(a0, a3)
    b1 = jnp.minimum(a1, a4)
    c1 = jnp.minimum(b1, z2)
    c2 = jnp.maximum(b1, z2)
    d2 = jnp.minimum(c2, b3)
    return jnp.maximum(c1, d2)


def _medpool(v):
    T = _T
    s0 = jnp.concatenate([v[:, 2:3], v[:, 1:2], v[:, :T - 2]], axis=1)
    s1 = jnp.concatenate([v[:, 1:2], v[:, :T - 1]], axis=1)
    s3 = jnp.concatenate([v[:, 1:], v[:, T - 2:T - 1]], axis=1)
    s4 = jnp.concatenate([v[:, 2:], v[:, T - 2:T - 1], v[:, T - 3:T - 2]],
                         axis=1)
    return _median5(s0, s1, v, s3, s4)


def _fused_body(x_ref, ut_ref, w_ref, b_ref, e_ref, post_ref, mask_ref):
    xb = x_ref[0]
    w8 = w_ref[...]
    h = jax.lax.dot_general(w8, xb, (((1,), (0,)), ((), ())),
                            preferred_element_type=jnp.float32)
    h = h + b_ref[...]
    z0 = h[0:1, :] / 10.0
    z1 = h[1:2, :] / 10.0
    m = jnp.maximum(z0, z1)
    e0 = jnp.exp(z0 - m)
    e1 = jnp.exp(z1 - m)
    s = e0 + e1
    p0 = e0 / s
    p1 = e1 / s
    post_ref[0, 0:1, :] = p0
    post_ref[0, 1:2, :] = p1
    eps = 1e-20
    l0 = jnp.log(p0)
    l1 = jnp.log(p1)
    u0 = ut_ref[0, 0:1, :]
    u1 = ut_ref[0, 1:2, :]
    g0 = -jnp.log(-jnp.log(u0 + eps) + eps)
    g1 = -jnp.log(-jnp.log(u1 + eps) + eps)
    zz0 = (l0 + g0) / 0.8
    zz1 = (l1 + g1) / 0.8
    mm = jnp.maximum(zz0, zz1)
    ee0 = jnp.exp(zz0 - mm)
    ee1 = jnp.exp(zz1 - mm)
    ss = ee0 + ee1
    y0 = ee0 / ss
    y1 = ee1 / ss
    selv = jnp.where(y1 > y0, 1.0, 0.0).astype(jnp.float32)
    v = (e_ref[0] * selv).astype(jnp.bfloat16)
    v = _medpool(v)
    v = _medpool(v)
    v = _medpool(v)
    mask_ref[0] = v.astype(jnp.float32)


def kernel(x, e, u, W, b):
    B, C, T = x.shape
    w8 = jnp.zeros((8, C), jnp.float32).at[:2].set(W)
    b8 = jnp.zeros((8, 1), jnp.float32).at[:2, 0].set(b)
    ut = jnp.transpose(u, (0, 2, 1))

    post_bt, mask = pl.pallas_call(
        _fused_body,
        grid=(B,),
        in_specs=[
            pl.BlockSpec((1, C, T), lambda i: (i, 0, 0)),
            pl.BlockSpec((1, 2, T), lambda i: (i, 0, 0)),
            pl.BlockSpec((8, C), lambda i: (0, 0)),
            pl.BlockSpec((8, 1), lambda i: (0, 0)),
            pl.BlockSpec((1, C, T), lambda i: (i, 0, 0)),
        ],
        out_specs=[
            pl.BlockSpec((1, 2, T), lambda i: (i, 0, 0)),
            pl.BlockSpec((1, C, T), lambda i: (i, 0, 0)),
        ],
        out_shape=[
            jax.ShapeDtypeStruct((B, 2, T), jnp.float32),
            jax.ShapeDtypeStruct((B, C, T), jnp.float32),
        ],
    )(x, ut, w8, b8, e)

    posterior = jnp.transpose(post_bt, (0, 2, 1))
    return posterior, mask
